# baseline (speedup 1.0000x reference)
import functools

import jax
import jax.numpy as jnp
from jax.experimental import pallas as pl
from jax.experimental.pallas import tpu as pltpu

N = 10000
G = 64
HID = 128
FIL = 128
NG = 50
T = 3
CUT = 10.0
K = 32
NC = 10

SMAX = 256
CH = 8
NCH = K // CH
NP = 2
P = 128

_LOG2 = 0.6931471805599453
_KC = 128.0
_STEP = CUT / (NG - 1)
_COEFF = -0.5 / (_STEP * _STEP)


def _fiota(shape, dim):
    return jax.lax.broadcasted_iota(jnp.int32, shape, dim).astype(jnp.float32)


def _dot(a, b):
    return jax.lax.dot_general(a, b, (((1,), (0,)), ((), ())),
                               preferred_element_type=jnp.float32)


def _ssp(v):
    return jnp.maximum(v, 0.0) + jnp.log(1.0 + jnp.exp(-jnp.abs(v))) - _LOG2


def _acos(x):
    u = jnp.abs(x)
    p = jnp.float32(-0.0012624911)
    for cc in (0.0066700901, -0.0170881256, 0.0308918810, -0.0501743046,
               0.0889789874, -0.2145988016, 1.5707963050):
        p = p * u + cc
    r = jnp.sqrt(jnp.maximum(1.0 - u, 0.0)) * p
    return jnp.where(x < 0.0, jnp.pi - r, r)


def _body(posz_ref, sz_ref, emb_ref, w1_ref, b1_ref, w2_ref, b2_ref,
          cfw1_ref, cfw2_ref, cfb2_ref, blkw_ref, blkb_ref,
          o1w_ref, o1b_ref, o2w_ref, o2b_ref,
          e1w_ref, e1b_ref, e2w_ref, e2b_ref,
          out_ref, acc_ref, tab_ref):
    f32 = jnp.float32
    g = pl.program_id(0)

    jio = _fiota((SMAX, SMAX), 1)
    iio = _fiota((SMAX, SMAX), 0)
    ji = jax.lax.broadcasted_iota(jnp.int32, (SMAX, SMAX), 1)
    eio = _fiota((SMAX, 100), 1)
    offs = _fiota((1, NG), 1) * _STEP

    @pl.when(g == 0)
    def _():
        acc_ref[...] = jnp.zeros_like(acc_ref)
        qio = _fiota((P, 1), 0)
        xq = jnp.cos((qio + 0.5) * (jnp.pi / P))
        dq = (0.5 * CUT) * (xq + 1.0)
        gq = jnp.exp(_COEFF * (dq - offs) ** 2)
        wcq = 0.5 * (jnp.cos(dq * (jnp.pi / CUT)) + 1.0)
        pio = _fiota((P, P), 0)
        qio2 = _fiota((P, P), 1)
        dmat = jnp.cos(pio * ((qio2 + 0.5) * (jnp.pi / P))) * (2.0 / P)
        dmat = jnp.where(pio == 0.0, 1.0 / P, dmat)
        for t in range(T):
            f1 = _ssp(_dot(gq, w1_ref[t]) + b1_ref[t])
            ft = (_dot(f1, w2_ref[t]) + b2_ref[t]) * wcq
            tab_ref[t] = _dot(dmat, ft)

    sz_s, h_s, cur_s = [], [], []
    for s in range(NP):
        pg = posz_ref[s]
        sz = sz_ref[s, 0, 0]
        posg = pg[:, 0:3]
        zg = pg[:, 3:4]

        h = _dot((eio == zg).astype(f32), emb_ref[...])

        sq = jnp.sum(posg * posg, axis=1, keepdims=True)
        dpp = jax.lax.dot_general(posg, posg, (((1,), (1,)), ((), ())),
                                  preferred_element_type=f32)
        sq_j = jax.lax.dot_general(jnp.ones((SMAX, 1), f32), sq,
                                   (((1,), (1,)), ((), ())),
                                   preferred_element_type=f32)
        d2 = jnp.maximum(sq + sq_j - 2.0 * dpp, 0.0)
        mask = (jio != iio) & (jio < sz) & (iio < sz)

        key = jnp.maximum(_KC - d2, 0.0)
        ki = jax.lax.bitcast_convert_type(key, jnp.int32)
        ki = (ki & jnp.int32(-256)) | (jnp.int32(SMAX - 1) - ji)
        sz_s.append(sz)
        h_s.append(h)
        cur_s.append(jnp.where(mask, jax.lax.bitcast_convert_type(ki, f32),
                               0.0))

    m_ls = [[] for _ in range(NP)]
    ohk_ls = [[] for _ in range(NP)]
    for _ in range(K):
        for s in range(NP):
            cur = cur_s[s]
            m = jnp.max(cur, axis=1, keepdims=True)
            sel = cur == m
            ohk_ls[s].append(sel.astype(f32))
            cur_s[s] = jnp.where(sel, 0.0, cur)
            m_ls[s].append(m)

    pio_l = _fiota((1, P), 1)
    van_s, oh_s, vc_s = [], [], []
    for s in range(NP):
        mm = jnp.concatenate(m_ls[s], axis=1)
        valid = mm > (_KC - CUT * CUT)
        de = jnp.where(valid, jnp.maximum(_KC - mm, 0.0), 1.0)
        distm = jnp.sqrt(jnp.maximum(de, 1e-12))
        xh = jnp.clip(distm * (2.0 / CUT) - 1.0, -1.0, 1.0)
        thm = _acos(xh)
        vcm = valid.astype(f32)
        van_l, oh_l, vc_l = [], [], []
        for c in range(NCH):
            ks = range(c * CH, (c + 1) * CH)
            th = jnp.concatenate([thm[:, k:k + 1] for k in ks], axis=0)
            van_l.append(jnp.cos(th * pio_l))
            oh_l.append(jnp.concatenate([ohk_ls[s][k] for k in ks], axis=0))
            vc_l.append(jnp.concatenate(
                [vcm[:, k:k + 1] for k in ks], axis=0))
        van_s.append(van_l)
        oh_s.append(oh_l)
        vc_s.append(vc_l)

    for t in range(T):
        xl_s = [_dot(h_s[s], cfw1_ref[t]) for s in range(NP)]
        agg_s = [jnp.zeros((SMAX, FIL), f32) for _ in range(NP)]
        for c in range(NCH):
            for s in range(NP):
                wf = _dot(van_s[s][c], tab_ref[t]) * vc_s[s][c]
                xg = _dot(oh_s[s][c], xl_s[s])
                msg = xg * wf
                agg = agg_s[s]
                for kk in range(CH):
                    agg = agg + msg[kk * SMAX:(kk + 1) * SMAX, :]
                agg_s[s] = agg
        for s in range(NP):
            hc = _dot(agg_s[s], cfw2_ref[t]) + cfb2_ref[t]
            h_s[s] = h_s[s] + _dot(_ssp(hc), blkw_ref[t]) + blkb_ref[t]

    gio = jax.lax.broadcasted_iota(jnp.int32, (G, 1), 0)
    upd = jnp.zeros((G, 1), f32)
    for s in range(NP):
        h2 = _ssp(_dot(h_s[s], o1w_ref[...]) + o1b_ref[...])
        pa = _dot(h2, o2w_ref[...]) + o2b_ref[...]
        rv = (_fiota((SMAX, 1), 0) < sz_s[s]).astype(f32)
        gsum = jnp.sum(pa * rv)
        upd = upd + jnp.where(gio == NP * g + s, gsum, 0.0)
    acc_ref[...] = acc_ref[...] + upd

    @pl.when(g == G // NP - 1)
    def _():
        accv = acc_ref[...]
        hid = jnp.maximum(accv * e1w_ref[...] + e1b_ref[...], 0.0)
        out_ref[...] = _dot(hid, e2w_ref[...]) + e2b_ref[...]


def _full(shape):
    nd = len(shape)
    return pl.BlockSpec(shape, lambda g, _nd=nd: (0,) * _nd)


@functools.partial(jax.jit, static_argnames=("interpret",))
def _run(posz, szf, emb, mlp_w1, mlp_b1, mlp_w2, mlp_b2, cf_w1, cf_w2, cf_b2,
         blk_w, blk_b, out1_w, out1_b, out2_w, out2_b,
         ext1_w, ext1_b, ext2_w, ext2_b, interpret=False):
    return pl.pallas_call(
        _body,
        grid=(G // NP,),
        in_specs=[
            pl.BlockSpec((NP, SMAX, 4), lambda g: (g, 0, 0)),
            pl.BlockSpec((NP, 1, 1), lambda g: (g, 0, 0)),
            _full((100, HID)),
            _full((T, NG, FIL)), _full((T, 1, FIL)),
            _full((T, FIL, FIL)), _full((T, 1, FIL)),
            _full((T, HID, FIL)),
            _full((T, FIL, HID)), _full((T, 1, HID)),
            _full((T, HID, HID)), _full((T, 1, HID)),
            _full((HID, HID // 2)), _full((1, HID // 2)),
            _full((HID // 2, 1)), _full((1, 1)),
            _full((1, HID)), _full((1, HID)),
            _full((HID, NC)), _full((1, NC)),
        ],
        out_specs=pl.BlockSpec((G, NC), lambda g: (0, 0)),
        out_shape=jax.ShapeDtypeStruct((G, NC), jnp.float32),
        scratch_shapes=[pltpu.VMEM((G, 1), jnp.float32),
                        pltpu.VMEM((T, P, FIL), jnp.float32)],
        interpret=interpret,
    )(posz, szf, emb, mlp_w1, mlp_b1, mlp_w2, mlp_b2, cf_w1, cf_w2, cf_b2,
      blk_w, blk_b, out1_w, out1_b, out2_w, out2_b,
      ext1_w, ext1_b, ext2_w, ext2_b)


def kernel(x, pos, batch, emb, mlp_w1, mlp_b1, mlp_w2, mlp_b2, cf_w1, cf_w2,
           cf_b2, blk_w, blk_b, out1_w, out1_b, out2_w, out2_b,
           ext1_w, ext1_b, ext2_w, ext2_b):
    starts = jnp.searchsorted(batch, jnp.arange(G + 1, dtype=batch.dtype))
    starts = starts.astype(jnp.int32)
    szf = (starts[1:] - starts[:-1]).astype(jnp.float32).reshape(G, 1, 1)
    idx = jnp.clip(starts[:-1, None] + jnp.arange(SMAX, dtype=jnp.int32)[None, :],
                   0, N - 1)
    posz = jnp.concatenate([pos, x.astype(jnp.float32)], axis=1)[idx]

    return _run(posz, szf, emb,
                mlp_w1, mlp_b1.reshape(T, 1, FIL),
                mlp_w2, mlp_b2.reshape(T, 1, FIL),
                cf_w1, cf_w2, cf_b2.reshape(T, 1, HID),
                blk_w, blk_b.reshape(T, 1, HID),
                out1_w, out1_b.reshape(1, HID // 2),
                out2_w, out2_b.reshape(1, 1),
                ext1_w, ext1_b.reshape(1, HID),
                ext2_w, ext2_b.reshape(1, NC))


# speedup vs baseline: 1.9859x; 1.9859x over previous
import functools

import jax
import jax.numpy as jnp
from jax.experimental import pallas as pl
from jax.experimental.pallas import tpu as pltpu

N = 10000
G = 64
HID = 128
FIL = 128
NG = 50
T = 3
CUT = 10.0
K = 32
NC = 10

SMAX = 256
CH = 8
NCH = K // CH
NP = 2

_LOG2 = 0.6931471805599453
_KC = 128.0
_STEP = CUT / (NG - 1)
_COEFF = -0.5 / (_STEP * _STEP)


def _fiota(shape, dim):
    return jax.lax.broadcasted_iota(jnp.int32, shape, dim).astype(jnp.float32)


def _dot(a, b):
    return jax.lax.dot_general(a, b, (((1,), (0,)), ((), ())),
                               preferred_element_type=jnp.float32)


def _ssp(v):
    return jnp.maximum(v, 0.0) + jnp.log(1.0 + jnp.exp(-jnp.abs(v))) - _LOG2


def _body(posz_ref, sz_ref, emb_ref, w1_ref, b1_ref, w2_ref, b2_ref,
          cfw1_ref, cfw2_ref, cfb2_ref, blkw_ref, blkb_ref,
          o1w_ref, o1b_ref, o2w_ref, o2b_ref,
          e1w_ref, e1b_ref, e2w_ref, e2b_ref,
          out_ref, acc_ref):
    f32 = jnp.float32
    g = pl.program_id(0)

    @pl.when(g == 0)
    def _():
        acc_ref[...] = jnp.zeros_like(acc_ref)

    jio = _fiota((SMAX, SMAX), 1)
    iio = _fiota((SMAX, SMAX), 0)
    ji = jax.lax.broadcasted_iota(jnp.int32, (SMAX, SMAX), 1)
    eio = _fiota((SMAX, 100), 1)
    offs = _fiota((1, NG), 1) * _STEP

    sz_s, h_s, cur_s = [], [], []
    for s in range(NP):
        pg = posz_ref[s]
        sz = sz_ref[s, 0, 0]
        posg = pg[:, 0:3]
        zg = pg[:, 3:4]

        h = _dot((eio == zg).astype(f32), emb_ref[...])

        sq = jnp.sum(posg * posg, axis=1, keepdims=True)
        dpp = jax.lax.dot_general(posg, posg, (((1,), (1,)), ((), ())),
                                  preferred_element_type=f32)
        sq_j = jax.lax.dot_general(jnp.ones((SMAX, 1), f32), sq,
                                   (((1,), (1,)), ((), ())),
                                   preferred_element_type=f32)
        d2 = jnp.maximum(sq + sq_j - 2.0 * dpp, 0.0)
        mask = (jio != iio) & (jio < sz) & (iio < sz)

        key = jnp.maximum(_KC - d2, 0.0)
        ki = jax.lax.bitcast_convert_type(key, jnp.int32)
        ki = (ki & jnp.int32(-256)) | (jnp.int32(SMAX - 1) - ji)
        sz_s.append(sz)
        h_s.append(h)
        cur_s.append(jnp.where(mask, jax.lax.bitcast_convert_type(ki, f32),
                               0.0))

    m_ls = [[] for _ in range(NP)]
    ohk_ls = [[] for _ in range(NP)]
    for _ in range(K):
        for s in range(NP):
            cur = cur_s[s]
            m = jnp.max(cur, axis=1, keepdims=True)
            sel = cur == m
            ohk_ls[s].append(sel.astype(f32))
            cur_s[s] = jnp.where(sel, 0.0, cur)
            m_ls[s].append(m)

    gs_s, oh_s, wc_s = [], [], []
    for s in range(NP):
        mm = jnp.concatenate(m_ls[s], axis=1)
        valid = mm > (_KC - CUT * CUT)
        de = jnp.where(valid, jnp.maximum(_KC - mm, 0.0), 1.0)
        distm = jnp.sqrt(jnp.maximum(de, 1e-12))
        wctm = jnp.where(valid,
                         0.5 * (jnp.cos(distm * (jnp.pi / CUT)) + 1.0), 0.0)
        gs_l, oh_l, wc_l = [], [], []
        for c in range(NCH):
            ks = range(c * CH, (c + 1) * CH)
            gs_l.append(jnp.concatenate(
                [jnp.exp(_COEFF * (distm[:, k:k + 1] - offs) ** 2)
                 for k in ks], axis=0))
            oh_l.append(jnp.concatenate([ohk_ls[s][k] for k in ks], axis=0))
            wc_l.append(jnp.concatenate(
                [wctm[:, k:k + 1] for k in ks], axis=0))
        gs_s.append(gs_l)
        oh_s.append(oh_l)
        wc_s.append(wc_l)

    for t in range(T):
        w1 = w1_ref[t]
        b1 = b1_ref[t]
        w2 = w2_ref[t]
        b2 = b2_ref[t]
        xl_s = [_dot(h_s[s], cfw1_ref[t]) for s in range(NP)]
        agg_s = [jnp.zeros((SMAX, FIL), f32) for _ in range(NP)]
        for c in range(NCH):
            for s in range(NP):
                f1 = _ssp(_dot(gs_s[s][c], w1) + b1)
                wf = (_dot(f1, w2) + b2) * wc_s[s][c]
                xg = _dot(oh_s[s][c], xl_s[s])
                msg = xg * wf
                agg = agg_s[s]
                for kk in range(CH):
                    agg = agg + msg[kk * SMAX:(kk + 1) * SMAX, :]
                agg_s[s] = agg
        for s in range(NP):
            hc = _dot(agg_s[s], cfw2_ref[t]) + cfb2_ref[t]
            h_s[s] = h_s[s] + _dot(_ssp(hc), blkw_ref[t]) + blkb_ref[t]

    gio = jax.lax.broadcasted_iota(jnp.int32, (G, 1), 0)
    upd = jnp.zeros((G, 1), f32)
    for s in range(NP):
        h2 = _ssp(_dot(h_s[s], o1w_ref[...]) + o1b_ref[...])
        pa = _dot(h2, o2w_ref[...]) + o2b_ref[...]
        rv = (_fiota((SMAX, 1), 0) < sz_s[s]).astype(f32)
        gsum = jnp.sum(pa * rv)
        upd = upd + jnp.where(gio == NP * g + s, gsum, 0.0)
    acc_ref[...] = acc_ref[...] + upd

    @pl.when(g == G // NP - 1)
    def _():
        accv = acc_ref[...]
        hid = jnp.maximum(accv * e1w_ref[...] + e1b_ref[...], 0.0)
        out_ref[...] = _dot(hid, e2w_ref[...]) + e2b_ref[...]


def _full(shape):
    nd = len(shape)
    return pl.BlockSpec(shape, lambda g, _nd=nd: (0,) * _nd)


@functools.partial(jax.jit, static_argnames=("interpret",))
def _run(posz, szf, emb, mlp_w1, mlp_b1, mlp_w2, mlp_b2, cf_w1, cf_w2, cf_b2,
         blk_w, blk_b, out1_w, out1_b, out2_w, out2_b,
         ext1_w, ext1_b, ext2_w, ext2_b, interpret=False):
    return pl.pallas_call(
        _body,
        grid=(G // NP,),
        in_specs=[
            pl.BlockSpec((NP, SMAX, 4), lambda g: (g, 0, 0)),
            pl.BlockSpec((NP, 1, 1), lambda g: (g, 0, 0)),
            _full((100, HID)),
            _full((T, NG, FIL)), _full((T, 1, FIL)),
            _full((T, FIL, FIL)), _full((T, 1, FIL)),
            _full((T, HID, FIL)),
            _full((T, FIL, HID)), _full((T, 1, HID)),
            _full((T, HID, HID)), _full((T, 1, HID)),
            _full((HID, HID // 2)), _full((1, HID // 2)),
            _full((HID // 2, 1)), _full((1, 1)),
            _full((1, HID)), _full((1, HID)),
            _full((HID, NC)), _full((1, NC)),
        ],
        out_specs=pl.BlockSpec((G, NC), lambda g: (0, 0)),
        out_shape=jax.ShapeDtypeStruct((G, NC), jnp.float32),
        scratch_shapes=[pltpu.VMEM((G, 1), jnp.float32)],
        interpret=interpret,
    )(posz, szf, emb, mlp_w1, mlp_b1, mlp_w2, mlp_b2, cf_w1, cf_w2, cf_b2,
      blk_w, blk_b, out1_w, out1_b, out2_w, out2_b,
      ext1_w, ext1_b, ext2_w, ext2_b)


def kernel(x, pos, batch, emb, mlp_w1, mlp_b1, mlp_w2, mlp_b2, cf_w1, cf_w2,
           cf_b2, blk_w, blk_b, out1_w, out1_b, out2_w, out2_b,
           ext1_w, ext1_b, ext2_w, ext2_b):
    starts = jnp.searchsorted(batch, jnp.arange(G + 1, dtype=batch.dtype))
    starts = starts.astype(jnp.int32)
    szf = (starts[1:] - starts[:-1]).astype(jnp.float32).reshape(G, 1, 1)
    idx = jnp.clip(starts[:-1, None] + jnp.arange(SMAX, dtype=jnp.int32)[None, :],
                   0, N - 1)
    posz = jnp.concatenate([pos, x.astype(jnp.float32)], axis=1)[idx]

    return _run(posz, szf, emb,
                mlp_w1, mlp_b1.reshape(T, 1, FIL),
                mlp_w2, mlp_b2.reshape(T, 1, FIL),
                cf_w1, cf_w2, cf_b2.reshape(T, 1, HID),
                blk_w, blk_b.reshape(T, 1, HID),
                out1_w, out1_b.reshape(1, HID // 2),
                out2_w, out2_b.reshape(1, 1),
                ext1_w, ext1_b.reshape(1, HID),
                ext2_w, ext2_b.reshape(1, NC))


# speedup vs baseline: 2.0110x; 1.0126x over previous
import functools

import jax
import jax.numpy as jnp
from jax.experimental import pallas as pl
from jax.experimental.pallas import tpu as pltpu

N = 10000
G = 64
HID = 128
FIL = 128
NG = 50
T = 3
CUT = 10.0
K = 32
NC = 10

SMAX = 256
CH = 8
NCH = K // CH
NP = 2

_LOG2 = 0.6931471805599453
_KC = 128.0
_STEP = CUT / (NG - 1)
_COEFF = -0.5 / (_STEP * _STEP)


def _fiota(shape, dim):
    return jax.lax.broadcasted_iota(jnp.int32, shape, dim).astype(jnp.float32)


def _dot(a, b):
    return jax.lax.dot_general(a, b, (((1,), (0,)), ((), ())),
                               preferred_element_type=jnp.float32)


def _ssp(v):
    return jnp.maximum(v, 0.0) + jnp.log(1.0 + jnp.exp(-jnp.abs(v))) - _LOG2


def _body(posz_ref, sz_ref, emb_ref, w1_ref, b1_ref, w2_ref, b2_ref,
          cfw1_ref, cfw2_ref, cfb2_ref, blkw_ref, blkb_ref,
          o1w_ref, o1b_ref, o2w_ref, o2b_ref,
          e1w_ref, e1b_ref, e2w_ref, e2b_ref,
          out_ref, acc_ref):
    f32 = jnp.float32
    g = pl.program_id(0)

    @pl.when(g == 0)
    def _():
        acc_ref[...] = jnp.zeros_like(acc_ref)

    jio = _fiota((SMAX, SMAX), 1)
    iio = _fiota((SMAX, SMAX), 0)
    ji = jax.lax.broadcasted_iota(jnp.int32, (SMAX, SMAX), 1)
    eio = _fiota((SMAX, 100), 1)
    offs = _fiota((1, NG), 1) * _STEP

    sz_s, h_s, cur_s = [], [], []
    for s in range(NP):
        pg = posz_ref[s]
        sz = sz_ref[s, 0, 0]
        posg = pg[:, 0:3]
        zg = pg[:, 3:4]

        h = _dot((eio == zg).astype(f32), emb_ref[...])

        sq = jnp.sum(posg * posg, axis=1, keepdims=True)
        dpp = jax.lax.dot_general(posg, posg, (((1,), (1,)), ((), ())),
                                  preferred_element_type=f32)
        sq_j = jax.lax.dot_general(jnp.ones((SMAX, 1), f32), sq,
                                   (((1,), (1,)), ((), ())),
                                   preferred_element_type=f32)
        d2 = jnp.maximum(sq + sq_j - 2.0 * dpp, 0.0)
        mask = (jio != iio) & (jio < sz) & (iio < sz)

        key = jnp.maximum(_KC - d2, 0.0)
        ki = jax.lax.bitcast_convert_type(key, jnp.int32)
        ki = (ki & jnp.int32(-256)) | (jnp.int32(SMAX - 1) - ji)
        sz_s.append(sz)
        h_s.append(h)
        cur_s.append(jnp.where(mask, jax.lax.bitcast_convert_type(ki, f32),
                               0.0))

    m_ls = [[] for _ in range(NP)]
    ohk_ls = [[] for _ in range(NP)]
    for _ in range(K):
        for s in range(NP):
            cur = cur_s[s]
            m = jnp.max(cur, axis=1, keepdims=True)
            sel = cur == m
            ohk_ls[s].append(sel.astype(jnp.bfloat16))
            cur_s[s] = jnp.where(sel, 0.0, cur)
            m_ls[s].append(m)

    gs_s, oh_s, wc_s = [], [], []
    for s in range(NP):
        mm = jnp.concatenate(m_ls[s], axis=1)
        valid = mm > (_KC - CUT * CUT)
        de = jnp.where(valid, jnp.maximum(_KC - mm, 0.0), 1.0)
        distm = jnp.sqrt(jnp.maximum(de, 1e-12))
        wctm = jnp.where(valid,
                         0.5 * (jnp.cos(distm * (jnp.pi / CUT)) + 1.0), 0.0)
        gs_l, oh_l, wc_l = [], [], []
        for c in range(NCH):
            ks = range(c * CH, (c + 1) * CH)
            gs_l.append(jnp.concatenate(
                [jnp.exp(_COEFF * (distm[:, k:k + 1] - offs) ** 2)
                 for k in ks], axis=0))
            oh_l.append(jnp.concatenate([ohk_ls[s][k] for k in ks], axis=0))
            wc_l.append(jnp.concatenate(
                [wctm[:, k:k + 1] for k in ks], axis=0))
        gs_s.append(gs_l)
        oh_s.append(oh_l)
        wc_s.append(wc_l)

    for t in range(T):
        w1 = w1_ref[t]
        b1 = b1_ref[t]
        w2 = w2_ref[t]
        b2 = b2_ref[t]
        xl_s = [_dot(h_s[s], cfw1_ref[t]).astype(jnp.bfloat16)
                for s in range(NP)]
        agg_s = [jnp.zeros((SMAX, FIL), f32) for _ in range(NP)]
        for c in range(NCH):
            for s in range(NP):
                f1 = _ssp(_dot(gs_s[s][c], w1) + b1)
                wf = (_dot(f1, w2) + b2) * wc_s[s][c]
                xg = _dot(oh_s[s][c], xl_s[s])
                msg = xg * wf
                agg = agg_s[s]
                for kk in range(CH):
                    agg = agg + msg[kk * SMAX:(kk + 1) * SMAX, :]
                agg_s[s] = agg
        for s in range(NP):
            hc = _dot(agg_s[s], cfw2_ref[t]) + cfb2_ref[t]
            h_s[s] = h_s[s] + _dot(_ssp(hc), blkw_ref[t]) + blkb_ref[t]

    gio = jax.lax.broadcasted_iota(jnp.int32, (G, 1), 0)
    upd = jnp.zeros((G, 1), f32)
    for s in range(NP):
        h2 = _ssp(_dot(h_s[s], o1w_ref[...]) + o1b_ref[...])
        pa = _dot(h2, o2w_ref[...]) + o2b_ref[...]
        rv = (_fiota((SMAX, 1), 0) < sz_s[s]).astype(f32)
        gsum = jnp.sum(pa * rv)
        upd = upd + jnp.where(gio == NP * g + s, gsum, 0.0)
    acc_ref[...] = acc_ref[...] + upd

    @pl.when(g == G // NP - 1)
    def _():
        accv = acc_ref[...]
        hid = jnp.maximum(accv * e1w_ref[...] + e1b_ref[...], 0.0)
        out_ref[...] = _dot(hid, e2w_ref[...]) + e2b_ref[...]


def _full(shape):
    nd = len(shape)
    return pl.BlockSpec(shape, lambda g, _nd=nd: (0,) * _nd)


@functools.partial(jax.jit, static_argnames=("interpret",))
def _run(posz, szf, emb, mlp_w1, mlp_b1, mlp_w2, mlp_b2, cf_w1, cf_w2, cf_b2,
         blk_w, blk_b, out1_w, out1_b, out2_w, out2_b,
         ext1_w, ext1_b, ext2_w, ext2_b, interpret=False):
    return pl.pallas_call(
        _body,
        grid=(G // NP,),
        in_specs=[
            pl.BlockSpec((NP, SMAX, 4), lambda g: (g, 0, 0)),
            pl.BlockSpec((NP, 1, 1), lambda g: (g, 0, 0)),
            _full((100, HID)),
            _full((T, NG, FIL)), _full((T, 1, FIL)),
            _full((T, FIL, FIL)), _full((T, 1, FIL)),
            _full((T, HID, FIL)),
            _full((T, FIL, HID)), _full((T, 1, HID)),
            _full((T, HID, HID)), _full((T, 1, HID)),
            _full((HID, HID // 2)), _full((1, HID // 2)),
            _full((HID // 2, 1)), _full((1, 1)),
            _full((1, HID)), _full((1, HID)),
            _full((HID, NC)), _full((1, NC)),
        ],
        out_specs=pl.BlockSpec((G, NC), lambda g: (0, 0)),
        out_shape=jax.ShapeDtypeStruct((G, NC), jnp.float32),
        scratch_shapes=[pltpu.VMEM((G, 1), jnp.float32)],
        interpret=interpret,
    )(posz, szf, emb, mlp_w1, mlp_b1, mlp_w2, mlp_b2, cf_w1, cf_w2, cf_b2,
      blk_w, blk_b, out1_w, out1_b, out2_w, out2_b,
      ext1_w, ext1_b, ext2_w, ext2_b)


def kernel(x, pos, batch, emb, mlp_w1, mlp_b1, mlp_w2, mlp_b2, cf_w1, cf_w2,
           cf_b2, blk_w, blk_b, out1_w, out1_b, out2_w, out2_b,
           ext1_w, ext1_b, ext2_w, ext2_b):
    starts = jnp.searchsorted(batch, jnp.arange(G + 1, dtype=batch.dtype))
    starts = starts.astype(jnp.int32)
    szf = (starts[1:] - starts[:-1]).astype(jnp.float32).reshape(G, 1, 1)
    idx = jnp.clip(starts[:-1, None] + jnp.arange(SMAX, dtype=jnp.int32)[None, :],
                   0, N - 1)
    posz = jnp.concatenate([pos, x.astype(jnp.float32)], axis=1)[idx]

    return _run(posz, szf, emb,
                mlp_w1, mlp_b1.reshape(T, 1, FIL),
                mlp_w2, mlp_b2.reshape(T, 1, FIL),
                cf_w1, cf_w2, cf_b2.reshape(T, 1, HID),
                blk_w, blk_b.reshape(T, 1, HID),
                out1_w, out1_b.reshape(1, HID // 2),
                out2_w, out2_b.reshape(1, 1),
                ext1_w, ext1_b.reshape(1, HID),
                ext2_w, ext2_b.reshape(1, NC))


# speedup vs baseline: 2.0160x; 1.0025x over previous
import functools

import jax
import jax.numpy as jnp
from jax.experimental import pallas as pl
from jax.experimental.pallas import tpu as pltpu

N = 10000
G = 64
HID = 128
FIL = 128
NG = 50
T = 3
CUT = 10.0
K = 32
NC = 10

SMAX = 256
CH = 8
NCH = K // CH
NP = 2

_LOG2 = 0.6931471805599453
_KC = 128.0
_STEP = CUT / (NG - 1)
_COEFF = -0.5 / (_STEP * _STEP)


def _fiota(shape, dim):
    return jax.lax.broadcasted_iota(jnp.int32, shape, dim).astype(jnp.float32)


def _dot(a, b):
    return jax.lax.dot_general(a, b, (((1,), (0,)), ((), ())),
                               preferred_element_type=jnp.float32)


def _ssp(v):
    return jnp.maximum(v, 0.0) + jnp.log(1.0 + jnp.exp(-jnp.abs(v))) - _LOG2


def _body(posz_ref, sz_ref, emb_ref, w1_ref, b1_ref, w2_ref, b2_ref,
          cfw1_ref, cfw2_ref, cfb2_ref, blkw_ref, blkb_ref,
          o1w_ref, o1b_ref, o2w_ref, o2b_ref,
          e1w_ref, e1b_ref, e2w_ref, e2b_ref,
          out_ref, acc_ref):
    f32 = jnp.float32
    g = pl.program_id(0)

    @pl.when(g == 0)
    def _():
        acc_ref[...] = jnp.zeros_like(acc_ref)

    jio = _fiota((SMAX, SMAX), 1)
    iio = _fiota((SMAX, SMAX), 0)
    ji = jax.lax.broadcasted_iota(jnp.int32, (SMAX, SMAX), 1)
    eio = _fiota((SMAX, 100), 1)
    offs = _fiota((1, NG), 1) * _STEP

    sz_s, h_s, cur_s = [], [], []
    for s in range(NP):
        pg = posz_ref[s]
        sz = sz_ref[s, 0, 0]
        posg = pg[:, 0:3]
        zg = pg[:, 3:4]

        h = _dot((eio == zg).astype(f32), emb_ref[...])

        sq = jnp.sum(posg * posg, axis=1, keepdims=True)
        dpp = jax.lax.dot_general(posg, posg, (((1,), (1,)), ((), ())),
                                  preferred_element_type=f32)
        sq_j = jax.lax.dot_general(jnp.ones((SMAX, 1), f32), sq,
                                   (((1,), (1,)), ((), ())),
                                   preferred_element_type=f32)
        d2 = jnp.maximum(sq + sq_j - 2.0 * dpp, 0.0)
        mask = (jio != iio) & (jio < sz) & (iio < sz)

        key = jnp.maximum(_KC - d2, 0.0)
        ki = jax.lax.bitcast_convert_type(key, jnp.int32)
        ki = (ki & jnp.int32(-256)) | (jnp.int32(SMAX - 1) - ji)
        sz_s.append(sz)
        h_s.append(h)
        cur_s.append(jnp.where(mask, jax.lax.bitcast_convert_type(ki, f32),
                               0.0))

    m_ls = [[] for _ in range(NP)]
    ohk_ls = [[] for _ in range(NP)]
    for _ in range(K):
        for s in range(NP):
            cur = cur_s[s]
            m = jnp.max(cur, axis=1, keepdims=True)
            sel = cur == m
            ohk_ls[s].append(sel.astype(jnp.bfloat16))
            cur_s[s] = jnp.where(sel, 0.0, cur)
            m_ls[s].append(m)

    gs_s, wc_s = [], []
    for s in range(NP):
        mm = jnp.concatenate(m_ls[s], axis=1)
        valid = mm > (_KC - CUT * CUT)
        de = jnp.where(valid, jnp.maximum(_KC - mm, 0.0), 1.0)
        distm = jnp.sqrt(jnp.maximum(de, 1e-12))
        wctm = jnp.where(valid,
                         0.5 * (jnp.cos(distm * (jnp.pi / CUT)) + 1.0), 0.0)
        gs_l, wc_l = [], []
        for c in range(NCH):
            ks = range(c * CH, (c + 1) * CH)
            gs_l.append(jnp.concatenate(
                [jnp.exp(_COEFF * (distm[:, k:k + 1] - offs) ** 2)
                 for k in ks], axis=0))
            wc_l.append(jnp.concatenate(
                [wctm[:, k:k + 1] for k in ks], axis=0))
        gs_s.append(gs_l)
        wc_s.append(wc_l)

    for t in range(T):
        w1 = w1_ref[t]
        b1 = b1_ref[t]
        w2 = w2_ref[t]
        b2 = b2_ref[t]
        xl_s = [_dot(h_s[s], cfw1_ref[t]).astype(jnp.bfloat16)
                for s in range(NP)]
        agg_s = [jnp.zeros((SMAX, FIL), f32) for _ in range(NP)]
        for c in range(NCH):
            for s in range(NP):
                f1 = _ssp(_dot(gs_s[s][c], w1) + b1)
                wf = (_dot(f1, w2) + b2) * wc_s[s][c]
                agg = agg_s[s]
                for kk in range(CH):
                    xg = _dot(ohk_ls[s][c * CH + kk], xl_s[s])
                    agg = agg + xg * wf[kk * SMAX:(kk + 1) * SMAX, :]
                agg_s[s] = agg
        for s in range(NP):
            hc = _dot(agg_s[s], cfw2_ref[t]) + cfb2_ref[t]
            h_s[s] = h_s[s] + _dot(_ssp(hc), blkw_ref[t]) + blkb_ref[t]

    gio = jax.lax.broadcasted_iota(jnp.int32, (G, 1), 0)
    upd = jnp.zeros((G, 1), f32)
    for s in range(NP):
        h2 = _ssp(_dot(h_s[s], o1w_ref[...]) + o1b_ref[...])
        pa = _dot(h2, o2w_ref[...]) + o2b_ref[...]
        rv = (_fiota((SMAX, 1), 0) < sz_s[s]).astype(f32)
        gsum = jnp.sum(pa * rv)
        upd = upd + jnp.where(gio == NP * g + s, gsum, 0.0)
    acc_ref[...] = acc_ref[...] + upd

    @pl.when(g == G // NP - 1)
    def _():
        accv = acc_ref[...]
        hid = jnp.maximum(accv * e1w_ref[...] + e1b_ref[...], 0.0)
        out_ref[...] = _dot(hid, e2w_ref[...]) + e2b_ref[...]


def _full(shape):
    nd = len(shape)
    return pl.BlockSpec(shape, lambda g, _nd=nd: (0,) * _nd)


@functools.partial(jax.jit, static_argnames=("interpret",))
def _run(posz, szf, emb, mlp_w1, mlp_b1, mlp_w2, mlp_b2, cf_w1, cf_w2, cf_b2,
         blk_w, blk_b, out1_w, out1_b, out2_w, out2_b,
         ext1_w, ext1_b, ext2_w, ext2_b, interpret=False):
    return pl.pallas_call(
        _body,
        grid=(G // NP,),
        in_specs=[
            pl.BlockSpec((NP, SMAX, 4), lambda g: (g, 0, 0)),
            pl.BlockSpec((NP, 1, 1), lambda g: (g, 0, 0)),
            _full((100, HID)),
            _full((T, NG, FIL)), _full((T, 1, FIL)),
            _full((T, FIL, FIL)), _full((T, 1, FIL)),
            _full((T, HID, FIL)),
            _full((T, FIL, HID)), _full((T, 1, HID)),
            _full((T, HID, HID)), _full((T, 1, HID)),
            _full((HID, HID // 2)), _full((1, HID // 2)),
            _full((HID // 2, 1)), _full((1, 1)),
            _full((1, HID)), _full((1, HID)),
            _full((HID, NC)), _full((1, NC)),
        ],
        out_specs=pl.BlockSpec((G, NC), lambda g: (0, 0)),
        out_shape=jax.ShapeDtypeStruct((G, NC), jnp.float32),
        scratch_shapes=[pltpu.VMEM((G, 1), jnp.float32)],
        interpret=interpret,
    )(posz, szf, emb, mlp_w1, mlp_b1, mlp_w2, mlp_b2, cf_w1, cf_w2, cf_b2,
      blk_w, blk_b, out1_w, out1_b, out2_w, out2_b,
      ext1_w, ext1_b, ext2_w, ext2_b)


def kernel(x, pos, batch, emb, mlp_w1, mlp_b1, mlp_w2, mlp_b2, cf_w1, cf_w2,
           cf_b2, blk_w, blk_b, out1_w, out1_b, out2_w, out2_b,
           ext1_w, ext1_b, ext2_w, ext2_b):
    starts = jnp.searchsorted(batch, jnp.arange(G + 1, dtype=batch.dtype))
    starts = starts.astype(jnp.int32)
    szf = (starts[1:] - starts[:-1]).astype(jnp.float32).reshape(G, 1, 1)
    idx = jnp.clip(starts[:-1, None] + jnp.arange(SMAX, dtype=jnp.int32)[None, :],
                   0, N - 1)
    posz = jnp.concatenate([pos, x.astype(jnp.float32)], axis=1)[idx]

    return _run(posz, szf, emb,
                mlp_w1, mlp_b1.reshape(T, 1, FIL),
                mlp_w2, mlp_b2.reshape(T, 1, FIL),
                cf_w1, cf_w2, cf_b2.reshape(T, 1, HID),
                blk_w, blk_b.reshape(T, 1, HID),
                out1_w, out1_b.reshape(1, HID // 2),
                out2_w, out2_b.reshape(1, 1),
                ext1_w, ext1_b.reshape(1, HID),
                ext2_w, ext2_b.reshape(1, NC))


# speedup vs baseline: 2.1161x; 1.0497x over previous
import functools

import jax
import jax.numpy as jnp
from jax.experimental import pallas as pl
from jax.experimental.pallas import tpu as pltpu

N = 10000
G = 64
HID = 128
FIL = 128
NG = 50
T = 3
CUT = 10.0
K = 32
NC = 10

SMAX = 256
CH = 4
NCH = K // CH
NP = 2

_LOG2 = 0.6931471805599453
_KC = 128.0
_STEP = CUT / (NG - 1)
_COEFF = -0.5 / (_STEP * _STEP)


def _fiota(shape, dim):
    return jax.lax.broadcasted_iota(jnp.int32, shape, dim).astype(jnp.float32)


def _dot(a, b):
    return jax.lax.dot_general(a, b, (((1,), (0,)), ((), ())),
                               preferred_element_type=jnp.float32)


def _ssp(v):
    return jnp.maximum(v, 0.0) + jnp.log(1.0 + jnp.exp(-jnp.abs(v))) - _LOG2


def _body(posz_ref, sz_ref, emb_ref, w1_ref, b1_ref, w2_ref, b2_ref,
          cfw1_ref, cfw2_ref, cfb2_ref, blkw_ref, blkb_ref,
          o1w_ref, o1b_ref, o2w_ref, o2b_ref,
          e1w_ref, e1b_ref, e2w_ref, e2b_ref,
          out_ref, acc_ref):
    f32 = jnp.float32
    g = pl.program_id(0)

    @pl.when(g == 0)
    def _():
        acc_ref[...] = jnp.zeros_like(acc_ref)

    jio = _fiota((SMAX, SMAX), 1)
    iio = _fiota((SMAX, SMAX), 0)
    ji = jax.lax.broadcasted_iota(jnp.int32, (SMAX, SMAX), 1)
    eio = _fiota((SMAX, 100), 1)
    offs = _fiota((1, NG), 1) * _STEP

    sz_s, h_s, cur_s = [], [], []
    for s in range(NP):
        pg = posz_ref[s]
        sz = sz_ref[s, 0, 0]
        posg = pg[:, 0:3]
        zg = pg[:, 3:4]

        h = _dot((eio == zg).astype(f32), emb_ref[...])

        sq = jnp.sum(posg * posg, axis=1, keepdims=True)
        dpp = jax.lax.dot_general(posg, posg, (((1,), (1,)), ((), ())),
                                  preferred_element_type=f32)
        sq_j = jax.lax.dot_general(jnp.ones((SMAX, 1), f32), sq,
                                   (((1,), (1,)), ((), ())),
                                   preferred_element_type=f32)
        d2 = jnp.maximum(sq + sq_j - 2.0 * dpp, 0.0)
        mask = (jio != iio) & (jio < sz) & (iio < sz)

        key = jnp.maximum(_KC - d2, 0.0)
        ki = jax.lax.bitcast_convert_type(key, jnp.int32)
        ki = (ki & jnp.int32(-256)) | (jnp.int32(SMAX - 1) - ji)
        sz_s.append(sz)
        h_s.append(h)
        cur_s.append(jnp.where(mask, jax.lax.bitcast_convert_type(ki, f32),
                               0.0))

    m_ls = [[] for _ in range(NP)]
    ohk_ls = [[] for _ in range(NP)]
    for _ in range(K):
        for s in range(NP):
            cur = cur_s[s]
            m = jnp.max(cur, axis=1, keepdims=True)
            sel = cur == m
            ohk_ls[s].append(sel.astype(jnp.bfloat16))
            cur_s[s] = jnp.where(sel, 0.0, cur)
            m_ls[s].append(m)

    gs_s, wc_s = [], []
    for s in range(NP):
        mm = jnp.concatenate(m_ls[s], axis=1)
        valid = mm > (_KC - CUT * CUT)
        de = jnp.where(valid, jnp.maximum(_KC - mm, 0.0), 1.0)
        distm = jnp.sqrt(jnp.maximum(de, 1e-12))
        wctm = jnp.where(valid,
                         0.5 * (jnp.cos(distm * (jnp.pi / CUT)) + 1.0), 0.0)
        gs_l, wc_l = [], []
        for c in range(NCH):
            ks = range(c * CH, (c + 1) * CH)
            gs_l.append(jnp.concatenate(
                [jnp.exp(_COEFF * (distm[:, k:k + 1] - offs) ** 2)
                 for k in ks], axis=0))
            wc_l.append(jnp.concatenate(
                [wctm[:, k:k + 1] for k in ks], axis=0))
        gs_s.append(gs_l)
        wc_s.append(wc_l)

    for t in range(T):
        w1 = w1_ref[t]
        b1 = b1_ref[t]
        w2 = w2_ref[t]
        b2 = b2_ref[t]
        xl_s = [_dot(h_s[s], cfw1_ref[t]).astype(jnp.bfloat16)
                for s in range(NP)]
        agg_s = [jnp.zeros((SMAX, FIL), f32) for _ in range(NP)]
        for c in range(NCH):
            for s in range(NP):
                f1 = _ssp(_dot(gs_s[s][c], w1) + b1)
                wf = (_dot(f1, w2) + b2) * wc_s[s][c]
                agg = agg_s[s]
                for kk in range(CH):
                    xg = _dot(ohk_ls[s][c * CH + kk], xl_s[s])
                    agg = agg + xg * wf[kk * SMAX:(kk + 1) * SMAX, :]
                agg_s[s] = agg
        for s in range(NP):
            hc = _dot(agg_s[s], cfw2_ref[t]) + cfb2_ref[t]
            h_s[s] = h_s[s] + _dot(_ssp(hc), blkw_ref[t]) + blkb_ref[t]

    gio = jax.lax.broadcasted_iota(jnp.int32, (G, 1), 0)
    upd = jnp.zeros((G, 1), f32)
    for s in range(NP):
        h2 = _ssp(_dot(h_s[s], o1w_ref[...]) + o1b_ref[...])
        pa = _dot(h2, o2w_ref[...]) + o2b_ref[...]
        rv = (_fiota((SMAX, 1), 0) < sz_s[s]).astype(f32)
        gsum = jnp.sum(pa * rv)
        upd = upd + jnp.where(gio == NP * g + s, gsum, 0.0)
    acc_ref[...] = acc_ref[...] + upd

    @pl.when(g == G // NP - 1)
    def _():
        accv = acc_ref[...]
        hid = jnp.maximum(accv * e1w_ref[...] + e1b_ref[...], 0.0)
        out_ref[...] = _dot(hid, e2w_ref[...]) + e2b_ref[...]


def _full(shape):
    nd = len(shape)
    return pl.BlockSpec(shape, lambda g, _nd=nd: (0,) * _nd)


@functools.partial(jax.jit, static_argnames=("interpret",))
def _run(posz, szf, emb, mlp_w1, mlp_b1, mlp_w2, mlp_b2, cf_w1, cf_w2, cf_b2,
         blk_w, blk_b, out1_w, out1_b, out2_w, out2_b,
         ext1_w, ext1_b, ext2_w, ext2_b, interpret=False):
    return pl.pallas_call(
        _body,
        grid=(G // NP,),
        in_specs=[
            pl.BlockSpec((NP, SMAX, 4), lambda g: (g, 0, 0)),
            pl.BlockSpec((NP, 1, 1), lambda g: (g, 0, 0)),
            _full((100, HID)),
            _full((T, NG, FIL)), _full((T, 1, FIL)),
            _full((T, FIL, FIL)), _full((T, 1, FIL)),
            _full((T, HID, FIL)),
            _full((T, FIL, HID)), _full((T, 1, HID)),
            _full((T, HID, HID)), _full((T, 1, HID)),
            _full((HID, HID // 2)), _full((1, HID // 2)),
            _full((HID // 2, 1)), _full((1, 1)),
            _full((1, HID)), _full((1, HID)),
            _full((HID, NC)), _full((1, NC)),
        ],
        out_specs=pl.BlockSpec((G, NC), lambda g: (0, 0)),
        out_shape=jax.ShapeDtypeStruct((G, NC), jnp.float32),
        scratch_shapes=[pltpu.VMEM((G, 1), jnp.float32)],
        interpret=interpret,
    )(posz, szf, emb, mlp_w1, mlp_b1, mlp_w2, mlp_b2, cf_w1, cf_w2, cf_b2,
      blk_w, blk_b, out1_w, out1_b, out2_w, out2_b,
      ext1_w, ext1_b, ext2_w, ext2_b)


def kernel(x, pos, batch, emb, mlp_w1, mlp_b1, mlp_w2, mlp_b2, cf_w1, cf_w2,
           cf_b2, blk_w, blk_b, out1_w, out1_b, out2_w, out2_b,
           ext1_w, ext1_b, ext2_w, ext2_b):
    starts = jnp.searchsorted(batch, jnp.arange(G + 1, dtype=batch.dtype))
    starts = starts.astype(jnp.int32)
    szf = (starts[1:] - starts[:-1]).astype(jnp.float32).reshape(G, 1, 1)
    idx = jnp.clip(starts[:-1, None] + jnp.arange(SMAX, dtype=jnp.int32)[None, :],
                   0, N - 1)
    posz = jnp.concatenate([pos, x.astype(jnp.float32)], axis=1)[idx]

    return _run(posz, szf, emb,
                mlp_w1, mlp_b1.reshape(T, 1, FIL),
                mlp_w2, mlp_b2.reshape(T, 1, FIL),
                cf_w1, cf_w2, cf_b2.reshape(T, 1, HID),
                blk_w, blk_b.reshape(T, 1, HID),
                out1_w, out1_b.reshape(1, HID // 2),
                out2_w, out2_b.reshape(1, 1),
                ext1_w, ext1_b.reshape(1, HID),
                ext2_w, ext2_b.reshape(1, NC))


# speedup vs baseline: 2.1475x; 1.0148x over previous
import functools

import jax
import jax.numpy as jnp
from jax.experimental import pallas as pl
from jax.experimental.pallas import tpu as pltpu

N = 10000
G = 64
HID = 128
FIL = 128
NG = 50
T = 3
CUT = 10.0
K = 32
NC = 10

SMAX = 256
CH = 2
NCH = K // CH
NP = 2

_LOG2 = 0.6931471805599453
_KC = 128.0
_STEP = CUT / (NG - 1)
_COEFF = -0.5 / (_STEP * _STEP)


def _fiota(shape, dim):
    return jax.lax.broadcasted_iota(jnp.int32, shape, dim).astype(jnp.float32)


def _dot(a, b):
    return jax.lax.dot_general(a, b, (((1,), (0,)), ((), ())),
                               preferred_element_type=jnp.float32)


def _ssp(v):
    return jnp.maximum(v, 0.0) + jnp.log(1.0 + jnp.exp(-jnp.abs(v))) - _LOG2


def _body(posz_ref, sz_ref, emb_ref, w1_ref, b1_ref, w2_ref, b2_ref,
          cfw1_ref, cfw2_ref, cfb2_ref, blkw_ref, blkb_ref,
          o1w_ref, o1b_ref, o2w_ref, o2b_ref,
          e1w_ref, e1b_ref, e2w_ref, e2b_ref,
          out_ref, acc_ref):
    f32 = jnp.float32
    g = pl.program_id(0)

    @pl.when(g == 0)
    def _():
        acc_ref[...] = jnp.zeros_like(acc_ref)

    jio = _fiota((SMAX, SMAX), 1)
    iio = _fiota((SMAX, SMAX), 0)
    ji = jax.lax.broadcasted_iota(jnp.int32, (SMAX, SMAX), 1)
    eio = _fiota((SMAX, 100), 1)
    offs = _fiota((1, NG), 1) * _STEP

    sz_s, h_s, cur_s = [], [], []
    for s in range(NP):
        pg = posz_ref[s]
        sz = sz_ref[s, 0, 0]
        posg = pg[:, 0:3]
        zg = pg[:, 3:4]

        h = _dot((eio == zg).astype(f32), emb_ref[...])

        sq = jnp.sum(posg * posg, axis=1, keepdims=True)
        dpp = jax.lax.dot_general(posg, posg, (((1,), (1,)), ((), ())),
                                  preferred_element_type=f32)
        sq_j = jax.lax.dot_general(jnp.ones((SMAX, 1), f32), sq,
                                   (((1,), (1,)), ((), ())),
                                   preferred_element_type=f32)
        d2 = jnp.maximum(sq + sq_j - 2.0 * dpp, 0.0)
        mask = (jio != iio) & (jio < sz) & (iio < sz)

        key = jnp.maximum(_KC - d2, 0.0)
        ki = jax.lax.bitcast_convert_type(key, jnp.int32)
        ki = (ki & jnp.int32(-256)) | (jnp.int32(SMAX - 1) - ji)
        sz_s.append(sz)
        h_s.append(h)
        cur_s.append(jnp.where(mask, jax.lax.bitcast_convert_type(ki, f32),
                               0.0))

    m_ls = [[] for _ in range(NP)]
    ohk_ls = [[] for _ in range(NP)]
    for _ in range(K):
        for s in range(NP):
            cur = cur_s[s]
            m = jnp.max(cur, axis=1, keepdims=True)
            sel = cur == m
            ohk_ls[s].append(sel.astype(jnp.bfloat16))
            cur_s[s] = jnp.where(sel, 0.0, cur)
            m_ls[s].append(m)

    gs_s, wc_s = [], []
    for s in range(NP):
        mm = jnp.concatenate(m_ls[s], axis=1)
        valid = mm > (_KC - CUT * CUT)
        de = jnp.where(valid, jnp.maximum(_KC - mm, 0.0), 1.0)
        distm = jnp.sqrt(jnp.maximum(de, 1e-12))
        wctm = jnp.where(valid,
                         0.5 * (jnp.cos(distm * (jnp.pi / CUT)) + 1.0), 0.0)
        gs_l, wc_l = [], []
        for c in range(NCH):
            ks = range(c * CH, (c + 1) * CH)
            gs_l.append(jnp.concatenate(
                [jnp.exp(_COEFF * (distm[:, k:k + 1] - offs) ** 2)
                 for k in ks], axis=0))
            wc_l.append(jnp.concatenate(
                [wctm[:, k:k + 1] for k in ks], axis=0))
        gs_s.append(gs_l)
        wc_s.append(wc_l)

    for t in range(T):
        w1 = w1_ref[t]
        b1 = b1_ref[t]
        w2 = w2_ref[t]
        b2 = b2_ref[t]
        xl_s = [_dot(h_s[s], cfw1_ref[t]).astype(jnp.bfloat16)
                for s in range(NP)]
        agg_s = [jnp.zeros((SMAX, FIL), f32) for _ in range(NP)]
        for c in range(NCH):
            for s in range(NP):
                f1 = _ssp(_dot(gs_s[s][c], w1) + b1)
                wf = (_dot(f1, w2) + b2) * wc_s[s][c]
                agg = agg_s[s]
                for kk in range(CH):
                    xg = _dot(ohk_ls[s][c * CH + kk], xl_s[s])
                    agg = agg + xg * wf[kk * SMAX:(kk + 1) * SMAX, :]
                agg_s[s] = agg
        for s in range(NP):
            hc = _dot(agg_s[s], cfw2_ref[t]) + cfb2_ref[t]
            h_s[s] = h_s[s] + _dot(_ssp(hc), blkw_ref[t]) + blkb_ref[t]

    gio = jax.lax.broadcasted_iota(jnp.int32, (G, 1), 0)
    upd = jnp.zeros((G, 1), f32)
    for s in range(NP):
        h2 = _ssp(_dot(h_s[s], o1w_ref[...]) + o1b_ref[...])
        pa = _dot(h2, o2w_ref[...]) + o2b_ref[...]
        rv = (_fiota((SMAX, 1), 0) < sz_s[s]).astype(f32)
        gsum = jnp.sum(pa * rv)
        upd = upd + jnp.where(gio == NP * g + s, gsum, 0.0)
    acc_ref[...] = acc_ref[...] + upd

    @pl.when(g == G // NP - 1)
    def _():
        accv = acc_ref[...]
        hid = jnp.maximum(accv * e1w_ref[...] + e1b_ref[...], 0.0)
        out_ref[...] = _dot(hid, e2w_ref[...]) + e2b_ref[...]


def _full(shape):
    nd = len(shape)
    return pl.BlockSpec(shape, lambda g, _nd=nd: (0,) * _nd)


@functools.partial(jax.jit, static_argnames=("interpret",))
def _run(posz, szf, emb, mlp_w1, mlp_b1, mlp_w2, mlp_b2, cf_w1, cf_w2, cf_b2,
         blk_w, blk_b, out1_w, out1_b, out2_w, out2_b,
         ext1_w, ext1_b, ext2_w, ext2_b, interpret=False):
    return pl.pallas_call(
        _body,
        grid=(G // NP,),
        in_specs=[
            pl.BlockSpec((NP, SMAX, 4), lambda g: (g, 0, 0)),
            pl.BlockSpec((NP, 1, 1), lambda g: (g, 0, 0)),
            _full((100, HID)),
            _full((T, NG, FIL)), _full((T, 1, FIL)),
            _full((T, FIL, FIL)), _full((T, 1, FIL)),
            _full((T, HID, FIL)),
            _full((T, FIL, HID)), _full((T, 1, HID)),
            _full((T, HID, HID)), _full((T, 1, HID)),
            _full((HID, HID // 2)), _full((1, HID // 2)),
            _full((HID // 2, 1)), _full((1, 1)),
            _full((1, HID)), _full((1, HID)),
            _full((HID, NC)), _full((1, NC)),
        ],
        out_specs=pl.BlockSpec((G, NC), lambda g: (0, 0)),
        out_shape=jax.ShapeDtypeStruct((G, NC), jnp.float32),
        scratch_shapes=[pltpu.VMEM((G, 1), jnp.float32)],
        interpret=interpret,
    )(posz, szf, emb, mlp_w1, mlp_b1, mlp_w2, mlp_b2, cf_w1, cf_w2, cf_b2,
      blk_w, blk_b, out1_w, out1_b, out2_w, out2_b,
      ext1_w, ext1_b, ext2_w, ext2_b)


def kernel(x, pos, batch, emb, mlp_w1, mlp_b1, mlp_w2, mlp_b2, cf_w1, cf_w2,
           cf_b2, blk_w, blk_b, out1_w, out1_b, out2_w, out2_b,
           ext1_w, ext1_b, ext2_w, ext2_b):
    starts = jnp.searchsorted(batch, jnp.arange(G + 1, dtype=batch.dtype))
    starts = starts.astype(jnp.int32)
    szf = (starts[1:] - starts[:-1]).astype(jnp.float32).reshape(G, 1, 1)
    idx = jnp.clip(starts[:-1, None] + jnp.arange(SMAX, dtype=jnp.int32)[None, :],
                   0, N - 1)
    posz = jnp.concatenate([pos, x.astype(jnp.float32)], axis=1)[idx]

    return _run(posz, szf, emb,
                mlp_w1, mlp_b1.reshape(T, 1, FIL),
                mlp_w2, mlp_b2.reshape(T, 1, FIL),
                cf_w1, cf_w2, cf_b2.reshape(T, 1, HID),
                blk_w, blk_b.reshape(T, 1, HID),
                out1_w, out1_b.reshape(1, HID // 2),
                out2_w, out2_b.reshape(1, 1),
                ext1_w, ext1_b.reshape(1, HID),
                ext2_w, ext2_b.reshape(1, NC))
